# baseline (speedup 1.0000x reference)
import functools

import jax
import jax.numpy as jnp
from jax import lax
from jax.experimental import pallas as pl
from jax.experimental.pallas import tpu as pltpu
from jax.experimental.pallas import tpu_sc as plsc

NC = 2
NS = 16
L = 16
CH = 128
DW = 8
QW = 64
NQ = 4


def _sc_aggregate(x4, src3, dst3, z2d, zd, ones_in, n_pad, n_chunks):
  rpt = n_pad // NS
  mesh = plsc.VectorSubcoreMesh(
      core_axis_name="c", subcore_axis_name="s", num_cores=NC, num_subcores=NS
  )

  d_split = n_chunks // 2

  @functools.partial(
      pl.kernel,
      out_type=(
          jax.ShapeDtypeStruct((NQ * n_pad, QW), jnp.float32),
          jax.ShapeDtypeStruct((NC * n_pad, DW), jnp.float32),
      ),
      mesh=mesh,
      compiler_params=pltpu.CompilerParams(use_tc_tiling_on_sc=False),
      scratch_types=[
          pltpu.VMEM((n_chunks, CH), jnp.int32),
          pltpu.VMEM((n_chunks, CH), jnp.int32),
          pltpu.VMEM((CH, QW), jnp.float32),
          pltpu.VMEM((CH, QW), jnp.float32),
          pltpu.VMEM((CH, DW), jnp.float32),
          pltpu.VMEM_SHARED((n_pad, QW), jnp.float32),
          pltpu.VMEM_SHARED((n_pad, DW), jnp.float32),
          pltpu.SemaphoreType.DMA,
          pltpu.SemaphoreType.DMA,
      ],
  )
  def agg_kernel(x4_hbm, src_hbm, dst_hbm, z2d_hbm, zd_hbm, ones_hbm, agg_out,
                 deg_out, src_v, dst_v, rows_a, rows_b, ones_v, acc_sh, deg_sh,
                 sem_a, sem_b):
    c = lax.axis_index("c")
    s = lax.axis_index("s")
    r0 = s * rpt

    pltpu.sync_copy(z2d_hbm.at[pl.ds(r0, rpt)], acc_sh.at[pl.ds(r0, rpt)])
    pltpu.sync_copy(zd_hbm.at[pl.ds(r0, rpt)], deg_sh.at[pl.ds(r0, rpt)])

    pltpu.sync_copy(src_hbm.at[s], src_v)
    pltpu.sync_copy(dst_hbm.at[s], dst_v)
    pltpu.sync_copy(ones_hbm, ones_v)

    def adj(i, _):
      j = i // (CH // L)
      g = i % (CH // L)
      v = src_v[j, pl.ds(g * L, L)]
      src_v[j, pl.ds(g * L, L)] = v * 4 + 2 * c
      return 0

    lax.fori_loop(0, n_chunks * (CH // L), adj, 0)

    plsc.subcore_barrier()

    def wait_a():
      pltpu.make_async_copy(x4_hbm.at[src_v.at[0]], rows_a, sem_a).wait()

    def wait_b():
      pltpu.make_async_copy(x4_hbm.at[src_v.at[0]], rows_b, sem_b).wait()

    def body2(h, _):
      j = 2 * h
      wait_a()
      pltpu.sync_copy(rows_a, acc_sh.at[dst_v.at[j]], add=True)
      pltpu.async_copy(x4_hbm.at[src_v.at[j + 2]], rows_a, sem_a)
      wait_b()
      pltpu.sync_copy(rows_b, acc_sh.at[dst_v.at[j + 1]], add=True)
      pltpu.async_copy(x4_hbm.at[src_v.at[j + 3]], rows_b, sem_b)
      return 0

    def dbody(j, _):
      pltpu.sync_copy(ones_v, deg_sh.at[dst_v.at[j]], add=True)
      return 0

    def bump(i, _):
      j = i // (CH // L)
      g = i % (CH // L)
      src_v[j, pl.ds(g * L, L)] = src_v[j, pl.ds(g * L, L)] + 1
      return 0

    for q in (0, 1):
      pltpu.async_copy(x4_hbm.at[src_v.at[0]], rows_a, sem_a)
      pltpu.async_copy(x4_hbm.at[src_v.at[1]], rows_b, sem_b)
      lax.fori_loop(0, n_chunks // 2 - 1, body2, 0)
      wait_a()
      pltpu.sync_copy(rows_a, acc_sh.at[dst_v.at[n_chunks - 2]], add=True)
      wait_b()
      pltpu.sync_copy(rows_b, acc_sh.at[dst_v.at[n_chunks - 1]], add=True)

      if q == 0:
        @pl.when(c == 0)
        def _():
          lax.fori_loop(0, d_split, dbody, 0)

        @pl.when(c == 1)
        def _():
          lax.fori_loop(d_split, n_chunks, dbody, 0)

      plsc.subcore_barrier()

      o0 = (2 * c + q) * n_pad + r0
      pltpu.sync_copy(acc_sh.at[pl.ds(r0, rpt)], agg_out.at[pl.ds(o0, rpt)])
      if q == 0:
        pltpu.sync_copy(z2d_hbm.at[pl.ds(r0, rpt)], acc_sh.at[pl.ds(r0, rpt)])
        lax.fori_loop(0, n_chunks * (CH // L), bump, 0)
        plsc.subcore_barrier()

    d0 = c * n_pad + r0
    pltpu.sync_copy(deg_sh.at[pl.ds(r0, rpt)], deg_out.at[pl.ds(d0, rpt)])

  return agg_kernel(x4, src3, dst3, z2d, zd, ones_in)


def _tc_combine(x, agg3, deg3, W_l, W_r, b_l2, n, f_in, f_out, R):
  def tc_body(x_ref, a0_ref, a1_ref, a2_ref, a3_ref, d0_ref, d1_ref, wl_ref,
              wr_ref, b_ref, o_ref):
    deg = d0_ref[0, :, 0:1] + d1_ref[0, :, 0:1]
    inv = 1.0 / jnp.maximum(deg, 1.0)
    mean = jnp.concatenate(
        [a0_ref[0], a1_ref[0], a2_ref[0], a3_ref[0]], axis=1) * inv
    acc = jnp.dot(mean, wl_ref[...], preferred_element_type=jnp.float32)
    acc = acc + jnp.dot(x_ref[...], wr_ref[...],
                        preferred_element_type=jnp.float32)
    o_ref[...] = jnp.maximum(acc + b_ref[...], 0.0)

  return pl.pallas_call(
      tc_body,
      grid=(n // R,),
      in_specs=[
          pl.BlockSpec((R, f_in), lambda i: (i, 0)),
          pl.BlockSpec((1, R, QW), lambda i: (0, i, 0)),
          pl.BlockSpec((1, R, QW), lambda i: (1, i, 0)),
          pl.BlockSpec((1, R, QW), lambda i: (2, i, 0)),
          pl.BlockSpec((1, R, QW), lambda i: (3, i, 0)),
          pl.BlockSpec((1, R, DW), lambda i: (0, i, 0)),
          pl.BlockSpec((1, R, DW), lambda i: (1, i, 0)),
          pl.BlockSpec((f_in, f_out), lambda i: (0, 0)),
          pl.BlockSpec((f_in, f_out), lambda i: (0, 0)),
          pl.BlockSpec((1, f_out), lambda i: (0, 0)),
      ],
      out_specs=pl.BlockSpec((R, f_out), lambda i: (i, 0)),
      out_shape=jax.ShapeDtypeStruct((n, f_out), jnp.float32),
  )(x, agg3, agg3, agg3, agg3, deg3, deg3, W_l, W_r, b_l2)


@jax.jit
def kernel(x, edge_index, W_l, b_l, W_r):
  n, f_in = x.shape
  e = edge_index.shape[1]
  f_out = W_l.shape[1]

  per_tile = -(-e // NS)
  n_chunks = 2 * -(-per_tile // (2 * CH))
  e_pad = n_chunks * CH * NS
  n_pad = -(-(n + 1) // (NS * 8)) * (NS * 8)

  src = edge_index[0]
  dst = edge_index[1]
  pad = e_pad - e
  src3 = jnp.concatenate([src, jnp.zeros((pad,), jnp.int32)]).reshape(
      NS, n_chunks, CH)
  dst3 = jnp.concatenate([dst, jnp.full((pad,), n, jnp.int32)]).reshape(
      NS, n_chunks, CH)

  x4 = x.reshape(NQ * n, QW)
  z2d = jnp.zeros((n_pad, QW), jnp.float32)
  zd = jnp.zeros((n_pad, DW), jnp.float32)
  ones_in = jnp.ones((CH, DW), jnp.float32)

  agg_flat, deg_flat = _sc_aggregate(x4, src3, dst3, z2d, zd, ones_in,
                                     n_pad, n_chunks)
  agg3 = agg_flat.reshape(NQ, n_pad, QW)
  deg3 = deg_flat.reshape(NC, n_pad, DW)

  b_l2 = b_l.reshape(1, f_out)
  R = 1000
  return _tc_combine(x, agg3, deg3, W_l, W_r, b_l2, n, f_in, f_out, R)


# speedup vs baseline: 1.0182x; 1.0182x over previous
import functools

import jax
import jax.numpy as jnp
from jax import lax
from jax.experimental import pallas as pl
from jax.experimental.pallas import tpu as pltpu
from jax.experimental.pallas import tpu_sc as plsc

NC = 2
NS = 16
L = 16
CH = 128
DW = 8
QW = 64
NQ = 4


def _sc_aggregate(x4, src3, dst3, z2d, zd, ones_in, n_pad, n_chunks):
  rpt = n_pad // NS
  mesh = plsc.VectorSubcoreMesh(
      core_axis_name="c", subcore_axis_name="s", num_cores=NC, num_subcores=NS
  )

  d_split = n_chunks // 2

  @functools.partial(
      pl.kernel,
      out_type=(
          jax.ShapeDtypeStruct((NQ * n_pad, QW), jnp.float32),
          jax.ShapeDtypeStruct((NC * n_pad, DW), jnp.float32),
      ),
      mesh=mesh,
      compiler_params=pltpu.CompilerParams(use_tc_tiling_on_sc=False),
      scratch_types=[
          pltpu.VMEM((n_chunks, CH), jnp.int32),
          pltpu.VMEM((n_chunks, CH), jnp.int32),
          [pltpu.VMEM((CH, QW), jnp.float32)] * 4,
          pltpu.VMEM((CH, DW), jnp.float32),
          pltpu.VMEM_SHARED((n_pad, QW), jnp.float32),
          pltpu.VMEM_SHARED((n_pad, DW), jnp.float32),
          [pltpu.SemaphoreType.DMA] * 4,
          [pltpu.SemaphoreType.DMA] * 4,
      ],
  )
  def agg_kernel(x4_hbm, src_hbm, dst_hbm, z2d_hbm, zd_hbm, ones_hbm, agg_out,
                 deg_out, src_v, dst_v, rows, ones_v, acc_sh, deg_sh,
                 gsem, ssem):
    c = lax.axis_index("c")
    s = lax.axis_index("s")
    r0 = s * rpt

    pltpu.sync_copy(z2d_hbm.at[pl.ds(r0, rpt)], acc_sh.at[pl.ds(r0, rpt)])
    pltpu.sync_copy(zd_hbm.at[pl.ds(r0, rpt)], deg_sh.at[pl.ds(r0, rpt)])

    pltpu.sync_copy(src_hbm.at[s], src_v)
    pltpu.sync_copy(dst_hbm.at[s], dst_v)
    pltpu.sync_copy(ones_hbm, ones_v)

    def adj(i, _):
      j = i // (CH // L)
      g = i % (CH // L)
      v = src_v[j, pl.ds(g * L, L)]
      src_v[j, pl.ds(g * L, L)] = v * 4 + 2 * c
      return 0

    lax.fori_loop(0, n_chunks * (CH // L), adj, 0)

    plsc.subcore_barrier()

    def wait_gather(b):
      pltpu.make_async_copy(x4_hbm.at[src_v.at[0]], rows[b], gsem[b]).wait()

    def wait_scatter(b):
      pltpu.make_async_copy(rows[b], acc_sh.at[dst_v.at[0]], ssem[b]).wait()

    def body4(h, _):
      j = 4 * h
      for b in range(4):
        wait_gather(b)
        pltpu.async_copy(rows[b], acc_sh.at[dst_v.at[j + b]], ssem[b],
                         add=True)
      for b in range(4):
        wait_scatter(b)
        pltpu.async_copy(x4_hbm.at[src_v.at[j + 4 + b]], rows[b], gsem[b])
      return 0

    def dbody(j, _):
      pltpu.sync_copy(ones_v, deg_sh.at[dst_v.at[j]], add=True)
      return 0

    def bump(i, _):
      j = i // (CH // L)
      g = i % (CH // L)
      src_v[j, pl.ds(g * L, L)] = src_v[j, pl.ds(g * L, L)] + 1
      return 0

    for q in (0, 1):
      for b in range(4):
        pltpu.async_copy(x4_hbm.at[src_v.at[b]], rows[b], gsem[b])
      lax.fori_loop(0, n_chunks // 4 - 1, body4, 0)
      for b in range(4):
        wait_gather(b)
        pltpu.async_copy(rows[b], acc_sh.at[dst_v.at[n_chunks - 4 + b]],
                         ssem[b], add=True)
      for b in range(4):
        wait_scatter(b)

      if q == 0:
        @pl.when(c == 0)
        def _():
          lax.fori_loop(0, d_split, dbody, 0)

        @pl.when(c == 1)
        def _():
          lax.fori_loop(d_split, n_chunks, dbody, 0)

      plsc.subcore_barrier()

      o0 = (2 * c + q) * n_pad + r0
      pltpu.sync_copy(acc_sh.at[pl.ds(r0, rpt)], agg_out.at[pl.ds(o0, rpt)])
      if q == 0:
        pltpu.sync_copy(z2d_hbm.at[pl.ds(r0, rpt)], acc_sh.at[pl.ds(r0, rpt)])
        lax.fori_loop(0, n_chunks * (CH // L), bump, 0)
        plsc.subcore_barrier()

    d0 = c * n_pad + r0
    pltpu.sync_copy(deg_sh.at[pl.ds(r0, rpt)], deg_out.at[pl.ds(d0, rpt)])

  return agg_kernel(x4, src3, dst3, z2d, zd, ones_in)


def _tc_combine(x, agg3, deg3, W_l, W_r, b_l2, n, f_in, f_out, R):
  def tc_body(x_ref, a0_ref, a1_ref, a2_ref, a3_ref, d0_ref, d1_ref, wl_ref,
              wr_ref, b_ref, o_ref):
    deg = d0_ref[0, :, 0:1] + d1_ref[0, :, 0:1]
    inv = 1.0 / jnp.maximum(deg, 1.0)
    mean = jnp.concatenate(
        [a0_ref[0], a1_ref[0], a2_ref[0], a3_ref[0]], axis=1) * inv
    acc = jnp.dot(mean, wl_ref[...], preferred_element_type=jnp.float32)
    acc = acc + jnp.dot(x_ref[...], wr_ref[...],
                        preferred_element_type=jnp.float32)
    o_ref[...] = jnp.maximum(acc + b_ref[...], 0.0)

  return pl.pallas_call(
      tc_body,
      grid=(n // R,),
      in_specs=[
          pl.BlockSpec((R, f_in), lambda i: (i, 0)),
          pl.BlockSpec((1, R, QW), lambda i: (0, i, 0)),
          pl.BlockSpec((1, R, QW), lambda i: (1, i, 0)),
          pl.BlockSpec((1, R, QW), lambda i: (2, i, 0)),
          pl.BlockSpec((1, R, QW), lambda i: (3, i, 0)),
          pl.BlockSpec((1, R, DW), lambda i: (0, i, 0)),
          pl.BlockSpec((1, R, DW), lambda i: (1, i, 0)),
          pl.BlockSpec((f_in, f_out), lambda i: (0, 0)),
          pl.BlockSpec((f_in, f_out), lambda i: (0, 0)),
          pl.BlockSpec((1, f_out), lambda i: (0, 0)),
      ],
      out_specs=pl.BlockSpec((R, f_out), lambda i: (i, 0)),
      out_shape=jax.ShapeDtypeStruct((n, f_out), jnp.float32),
  )(x, agg3, agg3, agg3, agg3, deg3, deg3, W_l, W_r, b_l2)


@jax.jit
def kernel(x, edge_index, W_l, b_l, W_r):
  n, f_in = x.shape
  e = edge_index.shape[1]
  f_out = W_l.shape[1]

  per_tile = -(-e // NS)
  n_chunks = 4 * -(-per_tile // (4 * CH))
  e_pad = n_chunks * CH * NS
  n_pad = -(-(n + 1) // (NS * 8)) * (NS * 8)

  src = edge_index[0]
  dst = edge_index[1]
  pad = e_pad - e
  src3 = jnp.concatenate([src, jnp.zeros((pad,), jnp.int32)]).reshape(
      NS, n_chunks, CH)
  dst3 = jnp.concatenate([dst, jnp.full((pad,), n, jnp.int32)]).reshape(
      NS, n_chunks, CH)

  x4 = x.reshape(NQ * n, QW)
  z2d = jnp.zeros((n_pad, QW), jnp.float32)
  zd = jnp.zeros((n_pad, DW), jnp.float32)
  ones_in = jnp.ones((CH, DW), jnp.float32)

  agg_flat, deg_flat = _sc_aggregate(x4, src3, dst3, z2d, zd, ones_in,
                                     n_pad, n_chunks)
  agg3 = agg_flat.reshape(NQ, n_pad, QW)
  deg3 = deg_flat.reshape(NC, n_pad, DW)

  b_l2 = b_l.reshape(1, f_out)
  R = 1000
  return _tc_combine(x, agg3, deg3, W_l, W_r, b_l2, n, f_in, f_out, R)


# speedup vs baseline: 1.3389x; 1.3149x over previous
import functools

import jax
import jax.numpy as jnp
from jax import lax
from jax.experimental import pallas as pl
from jax.experimental.pallas import tpu as pltpu
from jax.experimental.pallas import tpu_sc as plsc

NC = 2
NS = 16
L = 16
CH = 128
DW = 8
HW = 128


def _sc_aggregate(xh, src3, dst3, z2d, zd, ones_in, n_pad, n_chunks):
  rpt = n_pad // NS
  mesh = plsc.VectorSubcoreMesh(
      core_axis_name="c", subcore_axis_name="s", num_cores=NC, num_subcores=NS
  )

  d_split = n_chunks // 2

  @functools.partial(
      pl.kernel,
      out_type=(
          jax.ShapeDtypeStruct((NC * n_pad, HW), jnp.bfloat16),
          jax.ShapeDtypeStruct((NC * n_pad, DW), jnp.float32),
      ),
      mesh=mesh,
      compiler_params=pltpu.CompilerParams(use_tc_tiling_on_sc=False),
      scratch_types=[
          pltpu.VMEM((n_chunks, CH), jnp.int32),
          pltpu.VMEM((n_chunks, CH), jnp.int32),
          [pltpu.VMEM((CH, HW), jnp.bfloat16)] * 4,
          pltpu.VMEM((CH, DW), jnp.float32),
          pltpu.VMEM_SHARED((n_pad, HW), jnp.bfloat16),
          pltpu.VMEM_SHARED((n_pad, DW), jnp.float32),
          [pltpu.SemaphoreType.DMA] * 4,
          [pltpu.SemaphoreType.DMA] * 4,
          pltpu.SemaphoreType.DMA,
      ],
  )
  def agg_kernel(xh_hbm, src_hbm, dst_hbm, z2d_hbm, zd_hbm, ones_hbm, agg_out,
                 deg_out, src_v, dst_v, rows, ones_v, acc_sh, deg_sh,
                 gsem, ssem, dsem):
    c = lax.axis_index("c")
    s = lax.axis_index("s")
    r0 = s * rpt

    pltpu.sync_copy(z2d_hbm.at[pl.ds(r0, rpt)], acc_sh.at[pl.ds(r0, rpt)])
    pltpu.sync_copy(zd_hbm.at[pl.ds(r0, rpt)], deg_sh.at[pl.ds(r0, rpt)])

    pltpu.sync_copy(src_hbm.at[s], src_v)
    pltpu.sync_copy(dst_hbm.at[s], dst_v)
    pltpu.sync_copy(ones_hbm, ones_v)

    def adj(i, _):
      j = i // (CH // L)
      g = i % (CH // L)
      v = src_v[j, pl.ds(g * L, L)]
      src_v[j, pl.ds(g * L, L)] = v * 2 + c
      return 0

    lax.fori_loop(0, n_chunks * (CH // L), adj, 0)

    plsc.subcore_barrier()

    def wait_gather(b):
      pltpu.make_async_copy(xh_hbm.at[src_v.at[0]], rows[b], gsem[b]).wait()

    def wait_scatter(b):
      pltpu.make_async_copy(rows[b], acc_sh.at[dst_v.at[0]], ssem[b]).wait()

    def body4(h, _):
      j = 4 * h
      for b in range(4):
        wait_gather(b)
        pltpu.async_copy(rows[b], acc_sh.at[dst_v.at[j + b]], ssem[b],
                         add=True)
      for b in range(4):
        wait_scatter(b)
        pltpu.async_copy(xh_hbm.at[src_v.at[j + 4 + b]], rows[b], gsem[b])
      return 0

    for b in range(4):
      pltpu.async_copy(xh_hbm.at[src_v.at[b]], rows[b], gsem[b])
    lax.fori_loop(0, n_chunks // 4 - 1, body4, 0)
    for b in range(4):
      wait_gather(b)
      pltpu.async_copy(rows[b], acc_sh.at[dst_v.at[n_chunks - 4 + b]],
                       ssem[b], add=True)
    for b in range(4):
      wait_scatter(b)

    def dfire(j, _):
      pltpu.async_copy(ones_v, deg_sh.at[dst_v.at[j]], dsem, add=True)
      return 0

    def ddrain(j, _):
      pltpu.make_async_copy(ones_v, deg_sh.at[dst_v.at[0]], dsem).wait()
      return 0

    @pl.when(c == 0)
    def _():
      lax.fori_loop(0, d_split, dfire, 0)
      lax.fori_loop(0, d_split, ddrain, 0)

    @pl.when(c == 1)
    def _():
      lax.fori_loop(d_split, n_chunks, dfire, 0)
      lax.fori_loop(d_split, n_chunks, ddrain, 0)

    plsc.subcore_barrier()

    o0 = c * n_pad + r0
    pltpu.sync_copy(acc_sh.at[pl.ds(r0, rpt)], agg_out.at[pl.ds(o0, rpt)])
    pltpu.sync_copy(deg_sh.at[pl.ds(r0, rpt)], deg_out.at[pl.ds(o0, rpt)])

  return agg_kernel(xh, src3, dst3, z2d, zd, ones_in)


def _tc_combine(x, agg3, deg3, W_l, W_r, b_l2, n, f_in, f_out, R):
  def tc_body(x_ref, alo_ref, ahi_ref, d0_ref, d1_ref, wl_ref, wr_ref, b_ref,
              o_ref):
    deg = d0_ref[0, :, 0:1] + d1_ref[0, :, 0:1]
    inv = 1.0 / jnp.maximum(deg, 1.0)
    agg = jnp.concatenate([alo_ref[0], ahi_ref[0]], axis=1)
    mean = agg.astype(jnp.float32) * inv
    acc = jnp.dot(mean, wl_ref[...], preferred_element_type=jnp.float32)
    acc = acc + jnp.dot(x_ref[...], wr_ref[...],
                        preferred_element_type=jnp.float32)
    o_ref[...] = jnp.maximum(acc + b_ref[...], 0.0)

  return pl.pallas_call(
      tc_body,
      grid=(n // R,),
      in_specs=[
          pl.BlockSpec((R, f_in), lambda i: (i, 0)),
          pl.BlockSpec((1, R, HW), lambda i: (0, i, 0)),
          pl.BlockSpec((1, R, HW), lambda i: (1, i, 0)),
          pl.BlockSpec((1, R, DW), lambda i: (0, i, 0)),
          pl.BlockSpec((1, R, DW), lambda i: (1, i, 0)),
          pl.BlockSpec((f_in, f_out), lambda i: (0, 0)),
          pl.BlockSpec((f_in, f_out), lambda i: (0, 0)),
          pl.BlockSpec((1, f_out), lambda i: (0, 0)),
      ],
      out_specs=pl.BlockSpec((R, f_out), lambda i: (i, 0)),
      out_shape=jax.ShapeDtypeStruct((n, f_out), jnp.float32),
  )(x, agg3, agg3, deg3, deg3, W_l, W_r, b_l2)


@jax.jit
def kernel(x, edge_index, W_l, b_l, W_r):
  n, f_in = x.shape
  e = edge_index.shape[1]
  f_out = W_l.shape[1]

  per_tile = -(-e // NS)
  n_chunks = 4 * -(-per_tile // (4 * CH))
  e_pad = n_chunks * CH * NS
  n_pad = -(-(n + 1) // (NS * 8)) * (NS * 8)

  src = edge_index[0]
  dst = edge_index[1]
  pad = e_pad - e
  src3 = jnp.concatenate([src, jnp.zeros((pad,), jnp.int32)]).reshape(
      NS, n_chunks, CH)
  dst3 = jnp.concatenate([dst, jnp.full((pad,), n, jnp.int32)]).reshape(
      NS, n_chunks, CH)

  xh = x.astype(jnp.bfloat16).reshape(NC * n, HW)
  z2d = jnp.zeros((n_pad, HW), jnp.bfloat16)
  zd = jnp.zeros((n_pad, DW), jnp.float32)
  ones_in = jnp.ones((CH, DW), jnp.float32)

  agg_flat, deg_flat = _sc_aggregate(xh, src3, dst3, z2d, zd, ones_in,
                                     n_pad, n_chunks)
  agg3 = agg_flat.reshape(NC, n_pad, HW)
  deg3 = deg_flat.reshape(NC, n_pad, DW)

  b_l2 = b_l.reshape(1, f_out)
  R = 1000
  return _tc_combine(x, agg3, deg3, W_l, W_r, b_l2, n, f_in, f_out, R)
